# Initial kernel scaffold; baseline (speedup 1.0000x reference)
import jax
import jax.numpy as jnp
from jax.experimental import pallas as pl


def kernel(x_user, x_item, root_user, Wsrc1ui, Wdst1ui, asrc1ui, adst1ui, b1ui, Wsrc1iu, Wdst1iu, asrc1iu, adst1iu, b1iu, Wsrc2ui, Wdst2ui, asrc2ui, adst2ui, b2ui, Wsrc2iu, Wdst2iu, asrc2iu, adst2iu, b2iu, rel_weight, edge_index_ui, edge_index_iu, edge_label_index_ui, edge_label_index_iu):
    raise NotImplementedError("write your pallas kernel here")


# speedup vs baseline: 7.3782x; 7.3782x over previous
import functools

import jax
import jax.numpy as jnp
from jax import lax
from jax.experimental import pallas as pl
from jax.experimental.pallas import tpu as pltpu
from jax.experimental.pallas import tpu_sc as plsc

N = 10000
NP = 10240
D = 128
E = 160000
NW = 32
EPT = 5120
EP = NW * EPT
NB = EPT // 128
L = 50000
LBT = 13
LPT = LBT * 128
LP = NW * LPT

_F32 = jnp.float32
_I32 = jnp.int32


_RB = 400
_GRID = N // _RB


def _dotT(a1, m):
    return lax.dot_general(a1, m, (((1,), (1,)), ((), ())),
                           preferred_element_type=_F32)


def _col_dot(x, a1):
    col = lax.dot_general(x, a1, (((1,), (1,)), ((), ())),
                          preferred_element_type=_F32)
    return jnp.broadcast_to(col, (col.shape[0], 8))


def _prep1_body(xs_ref, xd_ref, ws_ref, as_ref, wd_ref, ad_ref,
                hs_ref, ss_ref, sd_ref):
    hs = jnp.dot(xs_ref[...], ws_ref[...], preferred_element_type=_F32)
    hs_ref[...] = hs
    ss_ref[...] = _col_dot(hs, as_ref[...])
    wda = _dotT(ad_ref[...], wd_ref[...])
    sd_ref[...] = _col_dot(xd_ref[...], wda)


def _prep1(x_src, x_dst, ws, a_s, wd, a_d):
    full = pl.BlockSpec((128, 128), lambda i: (0, 0))
    vec = pl.BlockSpec((1, 128), lambda i: (0, 0))
    blk = pl.BlockSpec((_RB, 128), lambda i: (i, 0))
    row = pl.BlockSpec((_RB, 8), lambda i: (i, 0))
    return pl.pallas_call(
        _prep1_body,
        grid=(_GRID,),
        in_specs=[blk, blk, full, vec, full, vec],
        out_specs=[blk, row, row],
        out_shape=[jax.ShapeDtypeStruct((N, D), _F32),
                   jax.ShapeDtypeStruct((N, 8), _F32),
                   jax.ShapeDtypeStruct((N, 8), _F32)],
    )(x_src, x_dst, ws, a_s.reshape(1, D), wd, a_d.reshape(1, D))


def _comb_body(n0_ref, n1_ref, d0_ref, d1_ref, b_ref, root_ref,
               ws_ref, as_ref, wd_ref, ad_ref, hs_ref, ss_ref, sd_ref):
    den = d0_ref[...] + d1_ref[...]
    item1 = (n0_ref[...] + n1_ref[...]) / (den + 1e-16) + b_ref[...]
    item1 = jnp.maximum(item1, 0.0)
    u1 = jnp.maximum(root_ref[...], 0.0)
    hs = jnp.dot(u1, ws_ref[...], preferred_element_type=_F32)
    hs_ref[...] = hs
    ss_ref[...] = _col_dot(hs, as_ref[...])
    wda = _dotT(ad_ref[...], wd_ref[...])
    sd_ref[...] = _col_dot(item1, wda)


def _comb_prep2(n0, n1, d0, d1, b, root, ws, a_s, wd, a_d):
    full = pl.BlockSpec((128, 128), lambda i: (0, 0))
    vec = pl.BlockSpec((1, 128), lambda i: (0, 0))
    blk = pl.BlockSpec((_RB, 128), lambda i: (i, 0))
    col = pl.BlockSpec((_RB, 1), lambda i: (i, 0))
    row = pl.BlockSpec((_RB, 8), lambda i: (i, 0))
    return pl.pallas_call(
        _comb_body,
        grid=(_GRID,),
        in_specs=[blk, blk, col, col, vec, blk, full, vec, full, vec],
        out_specs=[blk, row, row],
        out_shape=[jax.ShapeDtypeStruct((N, D), _F32),
                   jax.ShapeDtypeStruct((N, 8), _F32),
                   jax.ShapeDtypeStruct((N, 8), _F32)],
    )(n0, n1, d0, d1, b.reshape(1, D), root, ws, a_s.reshape(1, D),
      wd, a_d.reshape(1, D))


def _final_body(n0_ref, n1_ref, d0_ref, d1_ref, b_ref, root_ref,
                rw0_ref, rw1_ref, item2_ref, u2w_ref, i2w_ref):
    den = d0_ref[...] + d1_ref[...]
    item2 = (n0_ref[...] + n1_ref[...]) / (den + 1e-16) + b_ref[...]
    item2_ref[...] = item2
    u2w_ref[...] = root_ref[...] * rw0_ref[...]
    i2w_ref[...] = item2 * rw1_ref[...]


def _final(n0, n1, d0, d1, b, root, rw0, rw1):
    vec = pl.BlockSpec((1, 128), lambda i: (0, 0))
    blk = pl.BlockSpec((_RB, 128), lambda i: (i, 0))
    col = pl.BlockSpec((_RB, 1), lambda i: (i, 0))
    return pl.pallas_call(
        _final_body,
        grid=(_GRID,),
        in_specs=[blk, blk, col, col, vec, blk, vec, vec],
        out_specs=[blk, blk, blk],
        out_shape=[jax.ShapeDtypeStruct((N, D), _F32),
                   jax.ShapeDtypeStruct((N, D), _F32),
                   jax.ShapeDtypeStruct((N, D), _F32)],
    )(n0, n1, d0, d1, b.reshape(1, D), root, rw0, rw1)


@functools.lru_cache(maxsize=None)
def _edge_pass_kernel():
    mesh = plsc.VectorSubcoreMesh(core_axis_name="c", subcore_axis_name="s")
    return functools.partial(
        pl.kernel,
        mesh=mesh,
        out_type=[jax.ShapeDtypeStruct((2, NP, D), _F32),
                  jax.ShapeDtypeStruct((2 * NP,), _F32)],
        scratch_types=[
            pltpu.VMEM((128,), _F32),
            pltpu.VMEM((128,), _F32),
            pltpu.VMEM((EPT,), _I32),
            pltpu.VMEM((EPT,), _I32),
            pltpu.VMEM((NB, 128), _I32),
            pltpu.VMEM((128,), _F32),
            pltpu.VMEM((128, 128), _F32),
            pltpu.VMEM((40, 128), _F32),
            pltpu.VMEM((640,), _F32),
            pltpu.VMEM_SHARED((NP, D), _F32),
            pltpu.VMEM_SHARED((NP,), _F32),
            pltpu.SemaphoreType.DMA,
        ],
    )(_edge_pass_body)


def _edge_pass_body(hs_hbm, ss_hbm, sd_hbm, srcf_hbm, dst_hbm, dstf_hbm,
                    nump, denp,
                    ssg_v, sdg_v, src_f, dst_f, dst_v, w_v, rows_v, zrow_v,
                    zden_v, num_s, den_s, sem):
    c = lax.axis_index("c")
    s = lax.axis_index("s")
    wid = s * 2 + c
    tid = s

    pltpu.sync_copy(srcf_hbm.at[pl.ds(wid * EPT, EPT)], src_f)
    pltpu.sync_copy(dstf_hbm.at[pl.ds(wid * EPT, EPT)], dst_f)
    pltpu.sync_copy(dst_hbm.at[pl.ds(wid * NB, NB)], dst_v)

    z16 = jnp.zeros((16,), _F32)
    for r in range(40):
        for k in range(8):
            zrow_v[r, pl.ds(k * 16, 16)] = z16
    for i in range(40):
        zden_v[pl.ds(i * 16, 16)] = z16

    for j in range(16):
        pltpu.sync_copy(zrow_v, num_s.at[pl.ds(tid * 640 + j * 40, 40)])
    pltpu.sync_copy(zden_v, den_s.at[pl.ds(tid * 640, 640)])
    plsc.subcore_barrier()

    def batch_body(b, carry):
        bidx = src_f.at[pl.ds(b * 128, 128)]
        didx = dst_f.at[pl.ds(b * 128, 128)]
        cp_r = pltpu.async_copy(hs_hbm.at[bidx], rows_v, sem)
        cp_s = pltpu.async_copy(ss_hbm.at[bidx], ssg_v, sem)
        cp_d = pltpu.async_copy(sd_hbm.at[didx], sdg_v, sem)
        cp_r.wait()
        cp_s.wait()
        cp_d.wait()
        for g in range(8):
            x = ssg_v[pl.ds(g * 16, 16)] + sdg_v[pl.ds(g * 16, 16)]
            w16 = jnp.exp(jnp.maximum(x, x * 0.2))
            w_v[pl.ds(g * 16, 16)] = w16
            for el in range(16):
                e = g * 16 + el
                wv = jnp.full((16,), w16[el], _F32)
                for k in range(8):
                    rows_v[e, pl.ds(k * 16, 16)] = (
                        rows_v[e, pl.ds(k * 16, 16)] * wv)
        drow = dst_v.at[b]
        pltpu.sync_copy(w_v, den_s.at[drow], add=True)
        pltpu.sync_copy(rows_v, num_s.at[drow], add=True)
        return carry

    lax.fori_loop(0, NB, batch_body, 0)
    plsc.subcore_barrier()

    pltpu.sync_copy(num_s.at[pl.ds(tid * 640, 640)],
                    nump.at[c, pl.ds(tid * 640, 640)])
    pltpu.sync_copy(den_s.at[pl.ds(tid * 640, 640)],
                    denp.at[pl.ds(c * NP + tid * 640, 640)])


@functools.lru_cache(maxsize=None)
def _score_kernel():
    mesh = plsc.VectorSubcoreMesh(core_axis_name="c", subcore_axis_name="s")
    return functools.partial(
        pl.kernel,
        mesh=mesh,
        out_type=jax.ShapeDtypeStruct((2 * LP, 128), _F32),
        scratch_types=[
            pltpu.VMEM((LBT, 128), _I32),
            pltpu.VMEM((LBT, 128), _I32),
            pltpu.VMEM((128, 128), _F32),
            pltpu.VMEM((128, 128), _F32),
            pltpu.SemaphoreType.DMA,
            pltpu.SemaphoreType.DMA,
        ],
    )(_score_body)


def _score_body(u2w, item2, i2w, root, idx_a, idx_b, out,
                ia_v, ib_v, ra_v, rb_v, sem_a, sem_b):
    c = lax.axis_index("c")
    s = lax.axis_index("s")
    wid = s * 2 + c

    for r in range(2):
        tab_a = u2w if r == 0 else i2w
        tab_b = item2 if r == 0 else root
        pltpu.sync_copy(idx_a.at[r, wid], ia_v)
        pltpu.sync_copy(idx_b.at[r, wid], ib_v)

        def bb_body(bb, carry, tab_a=tab_a, tab_b=tab_b, r=r):
            cp_a = pltpu.async_copy(tab_a.at[ia_v.at[bb]], ra_v, sem_a)
            cp_b = pltpu.async_copy(tab_b.at[ib_v.at[bb]], rb_v, sem_b)
            cp_a.wait()
            cp_b.wait()
            for e in range(128):
                for k in range(8):
                    ra_v[e, pl.ds(k * 16, 16)] = (
                        ra_v[e, pl.ds(k * 16, 16)] *
                        rb_v[e, pl.ds(k * 16, 16)])
            pltpu.sync_copy(
                ra_v, out.at[pl.ds(r * LP + wid * LPT + bb * 128, 128)])
            return carry

        lax.fori_loop(0, LBT, bb_body, 0)


def _rowsum_body(x_ref, o_ref):
    s = jnp.sum(x_ref[...], axis=1, keepdims=True)
    o_ref[...] = jnp.broadcast_to(s, (s.shape[0], 8))


def _rowsum(x):
    rows = x.shape[0]
    rb = 512
    return pl.pallas_call(
        _rowsum_body,
        grid=(rows // rb,),
        in_specs=[pl.BlockSpec((rb, 128), lambda i: (i, 0))],
        out_specs=pl.BlockSpec((rb, 8), lambda i: (i, 0)),
        out_shape=jax.ShapeDtypeStruct((rows, 8), _F32),
    )(x)


def kernel(x_user, x_item, root_user,
           Wsrc1ui, Wdst1ui, asrc1ui, adst1ui, b1ui,
           Wsrc1iu, Wdst1iu, asrc1iu, adst1iu, b1iu,
           Wsrc2ui, Wdst2ui, asrc2ui, adst2ui, b2ui,
           Wsrc2iu, Wdst2iu, asrc2iu, adst2iu, b2iu,
           rel_weight,
           edge_index_ui, edge_index_iu,
           edge_label_index_ui, edge_label_index_iu):
    src = edge_index_ui[0].astype(_I32)
    dst = edge_index_ui[1].astype(_I32)
    src_f = jnp.concatenate([src, jnp.zeros((EP - E,), _I32)])
    dst_f = jnp.concatenate([dst, jnp.full((EP - E,), N, _I32)])
    dst_m = dst_f.reshape(-1, 128)

    hs1, ss1, sd1 = _prep1(x_user, x_item, Wsrc1ui, asrc1ui, Wdst1ui, adst1ui)
    zpad = jnp.zeros((NP - N,), _F32)
    nump1, denp1 = _edge_pass_kernel()(
        hs1, ss1[:, 0], jnp.concatenate([sd1[:, 0], zpad]),
        src_f, dst_m, dst_f)
    den1 = denp1.reshape(2, NP)[:, :N]
    nump1 = nump1[:, :N]

    hs2, ss2, sd2 = _comb_prep2(
        nump1[0], nump1[1], den1[0].reshape(N, 1), den1[1].reshape(N, 1),
        b1ui, root_user, Wsrc2ui, asrc2ui, Wdst2ui, adst2ui)
    nump2, denp2 = _edge_pass_kernel()(
        hs2, ss2[:, 0], jnp.concatenate([sd2[:, 0], zpad]),
        src_f, dst_m, dst_f)
    den2 = denp2.reshape(2, NP)[:, :N]
    nump2 = nump2[:, :N]

    item2, u2w, i2w = _final(
        nump2[0], nump2[1], den2[0].reshape(N, 1), den2[1].reshape(N, 1),
        b2ui, root_user, rel_weight[0].reshape(1, D), rel_weight[1].reshape(1, D))

    def _pad(a):
        return jnp.concatenate([a.astype(_I32), jnp.zeros((LP - L,), _I32)])

    idx_a = jnp.stack([_pad(edge_label_index_ui[0]),
                       _pad(edge_label_index_iu[0])]).reshape(2, NW, LBT, 128)
    idx_b = jnp.stack([_pad(edge_label_index_ui[1]),
                       _pad(edge_label_index_iu[1])]).reshape(2, NW, LBT, 128)

    prod = _score_kernel()(u2w, item2, i2w, root_user, idx_a, idx_b)
    preds = _rowsum(prod)[:, 0]
    return preds.reshape(2, LP)[:, :L]


# speedup vs baseline: 8.3255x; 1.1284x over previous
import functools

import jax
import jax.numpy as jnp
from jax import lax
from jax.experimental import pallas as pl
from jax.experimental.pallas import tpu as pltpu
from jax.experimental.pallas import tpu_sc as plsc

N = 10000
NP = 10240
NPS = 10112
NPT = NPS // 16
D = 128
E = 160000
NW = 32
EPT = 5120
EP = NW * EPT
NB = EPT // 128
L = 50000
LBT = 13
LPT = LBT * 128
LP = NW * LPT

_F32 = jnp.float32
_I32 = jnp.int32


_RB = 400
_GRID = N // _RB


def _dotT(a1, m):
    return lax.dot_general(a1, m, (((1,), (1,)), ((), ())),
                           preferred_element_type=_F32)


def _col_dot(x, a1):
    col = lax.dot_general(x, a1, (((1,), (1,)), ((), ())),
                          preferred_element_type=_F32)
    return jnp.broadcast_to(col, (col.shape[0], 8))


def _prep1_body(xs_ref, xd_ref, ws_ref, as_ref, wd_ref, ad_ref,
                hs_ref, ss_ref, sd_ref):
    hs = jnp.dot(xs_ref[...], ws_ref[...], preferred_element_type=_F32)
    hs_ref[...] = hs
    ss_ref[...] = _col_dot(hs, as_ref[...])
    wda = _dotT(ad_ref[...], wd_ref[...])
    sd_ref[...] = _col_dot(xd_ref[...], wda)


def _prep1(x_src, x_dst, ws, a_s, wd, a_d):
    full = pl.BlockSpec((128, 128), lambda i: (0, 0))
    vec = pl.BlockSpec((1, 128), lambda i: (0, 0))
    blk = pl.BlockSpec((_RB, 128), lambda i: (i, 0))
    row = pl.BlockSpec((_RB, 8), lambda i: (i, 0))
    return pl.pallas_call(
        _prep1_body,
        grid=(_GRID,),
        in_specs=[blk, blk, full, vec, full, vec],
        out_specs=[blk, row, row],
        out_shape=[jax.ShapeDtypeStruct((N, D), _F32),
                   jax.ShapeDtypeStruct((N, 8), _F32),
                   jax.ShapeDtypeStruct((N, 8), _F32)],
    )(x_src, x_dst, ws, a_s.reshape(1, D), wd, a_d.reshape(1, D))


def _comb_body(n0_ref, n1_ref, d0_ref, d1_ref, b_ref, root_ref,
               ws_ref, as_ref, wd_ref, ad_ref, hs_ref, ss_ref, sd_ref):
    den = d0_ref[...] + d1_ref[...]
    item1 = (n0_ref[...] + n1_ref[...]) / (den + 1e-16) + b_ref[...]
    item1 = jnp.maximum(item1, 0.0)
    u1 = jnp.maximum(root_ref[...], 0.0)
    hs = jnp.dot(u1, ws_ref[...], preferred_element_type=_F32)
    hs_ref[...] = hs
    ss_ref[...] = _col_dot(hs, as_ref[...])
    wda = _dotT(ad_ref[...], wd_ref[...])
    sd_ref[...] = _col_dot(item1, wda)


def _comb_prep2(n0, n1, d0, d1, b, root, ws, a_s, wd, a_d):
    full = pl.BlockSpec((128, 128), lambda i: (0, 0))
    vec = pl.BlockSpec((1, 128), lambda i: (0, 0))
    blk = pl.BlockSpec((_RB, 128), lambda i: (i, 0))
    col = pl.BlockSpec((_RB, 1), lambda i: (i, 0))
    row = pl.BlockSpec((_RB, 8), lambda i: (i, 0))
    return pl.pallas_call(
        _comb_body,
        grid=(_GRID,),
        in_specs=[blk, blk, col, col, vec, blk, full, vec, full, vec],
        out_specs=[blk, row, row],
        out_shape=[jax.ShapeDtypeStruct((N, D), _F32),
                   jax.ShapeDtypeStruct((N, 8), _F32),
                   jax.ShapeDtypeStruct((N, 8), _F32)],
    )(n0, n1, d0, d1, b.reshape(1, D), root, ws, a_s.reshape(1, D),
      wd, a_d.reshape(1, D))


def _final_body(n0_ref, n1_ref, d0_ref, d1_ref, b_ref, root_ref,
                rw0_ref, rw1_ref, item2_ref, u2w_ref, i2w_ref):
    den = d0_ref[...] + d1_ref[...]
    item2 = (n0_ref[...] + n1_ref[...]) / (den + 1e-16) + b_ref[...]
    item2_ref[...] = item2
    u2w_ref[...] = root_ref[...] * rw0_ref[...]
    i2w_ref[...] = item2 * rw1_ref[...]


def _final(n0, n1, d0, d1, b, root, rw0, rw1):
    vec = pl.BlockSpec((1, 128), lambda i: (0, 0))
    blk = pl.BlockSpec((_RB, 128), lambda i: (i, 0))
    col = pl.BlockSpec((_RB, 1), lambda i: (i, 0))
    return pl.pallas_call(
        _final_body,
        grid=(_GRID,),
        in_specs=[blk, blk, col, col, vec, blk, vec, vec],
        out_specs=[blk, blk, blk],
        out_shape=[jax.ShapeDtypeStruct((N, D), _F32),
                   jax.ShapeDtypeStruct((N, D), _F32),
                   jax.ShapeDtypeStruct((N, D), _F32)],
    )(n0, n1, d0, d1, b.reshape(1, D), root, rw0, rw1)


@functools.lru_cache(maxsize=None)
def _edge_pass_kernel():
    mesh = plsc.VectorSubcoreMesh(core_axis_name="c", subcore_axis_name="s")
    return functools.partial(
        pl.kernel,
        mesh=mesh,
        out_type=[jax.ShapeDtypeStruct((2, NPS, D), _F32),
                  jax.ShapeDtypeStruct((2 * NP,), _F32)],
        scratch_types=[
            pltpu.VMEM((128,), _F32),
            pltpu.VMEM((128,), _F32),
            pltpu.VMEM((128,), _F32),
            pltpu.VMEM((128,), _F32),
            pltpu.VMEM((EPT,), _I32),
            pltpu.VMEM((EPT,), _I32),
            pltpu.VMEM((NB, 128), _I32),
            pltpu.VMEM((128,), _F32),
            pltpu.VMEM((128, 128), _F32),
            pltpu.VMEM((128, 128), _F32),
            pltpu.VMEM((640,), _F32),
            pltpu.VMEM_SHARED((NPS, D), _F32),
            pltpu.VMEM_SHARED((NP,), _F32),
            pltpu.SemaphoreType.DMA,
            pltpu.SemaphoreType.DMA,
        ],
    )(_edge_pass_body)


def _edge_pass_body(hs_hbm, ss_hbm, sd_hbm, srcf_hbm, dst_hbm, dstf_hbm,
                    nump, denp,
                    ssg0, ssg1, sdg0, sdg1, src_f, dst_f, dst_v, w_v,
                    rows0, rows1, zden_v, num_s, den_s, sem0, sem1):
    c = lax.axis_index("c")
    s = lax.axis_index("s")
    wid = s * 2 + c
    tid = s

    ssg = (ssg0, ssg1)
    sdg = (sdg0, sdg1)
    rows = (rows0, rows1)
    sem = (sem0, sem1)

    pltpu.sync_copy(srcf_hbm.at[pl.ds(wid * EPT, EPT)], src_f)
    pltpu.sync_copy(dstf_hbm.at[pl.ds(wid * EPT, EPT)], dst_f)
    pltpu.sync_copy(dst_hbm.at[pl.ds(wid * NB, NB)], dst_v)

    z16 = jnp.zeros((16,), _F32)

    def zrow_body(r, carry):
        for k in range(8):
            rows0[r, pl.ds(k * 16, 16)] = z16
        return carry

    lax.fori_loop(0, 128, zrow_body, 0)
    for i in range(40):
        zden_v[pl.ds(i * 16, 16)] = z16

    for j in range(4):
        pltpu.sync_copy(rows0, num_s.at[pl.ds(tid * NPT + j * 128, 128)])
    pltpu.sync_copy(rows0.at[pl.ds(0, NPT - 512)],
                    num_s.at[pl.ds(tid * NPT + 512, NPT - 512)])
    pltpu.sync_copy(zden_v, den_s.at[pl.ds(tid * 640, 640)])
    plsc.subcore_barrier()

    def _gathers(b, p):
        bidx = src_f.at[pl.ds(b * 128, 128)]
        didx = dst_f.at[pl.ds(b * 128, 128)]
        return ((hs_hbm.at[bidx], rows[p], sem[p]),
                (ss_hbm.at[bidx], ssg[p], sem[p]),
                (sd_hbm.at[didx], sdg[p], sem[p]))

    def issue(b, p):
        for a in _gathers(b, p):
            pltpu.async_copy(*a)

    def drain(b, p):
        for a in _gathers(b, p):
            pltpu.make_async_copy(*a).wait()

    def process(b, p):
        def g_body(g, carry, p=p):
            x = ssg[p][pl.ds(g * 16, 16)] + sdg[p][pl.ds(g * 16, 16)]
            w16 = jnp.exp(jnp.maximum(x, x * 0.2))
            w_v[pl.ds(g * 16, 16)] = w16
            for el in range(16):
                e = g * 16 + el
                wv = jnp.full((16,), w16[el], _F32)
                for k in range(8):
                    rows[p][e, pl.ds(k * 16, 16)] = (
                        rows[p][e, pl.ds(k * 16, 16)] * wv)
            return carry

        lax.fori_loop(0, 8, g_body, 0)
        drow = dst_v.at[b]
        pltpu.sync_copy(w_v, den_s.at[drow], add=True)
        pltpu.sync_copy(rows[p], num_s.at[drow], add=True)

    issue(0, 0)

    def dbl_body(i, carry):
        b0 = i * 2
        drain(b0, 0)
        issue(b0 + 1, 1)
        process(b0, 0)
        drain(b0 + 1, 1)

        @pl.when(i < NB // 2 - 1)
        def _():
            issue(b0 + 2, 0)

        process(b0 + 1, 1)
        return carry

    lax.fori_loop(0, NB // 2, dbl_body, 0)
    plsc.subcore_barrier()

    pltpu.sync_copy(num_s.at[pl.ds(tid * NPT, NPT)],
                    nump.at[c, pl.ds(tid * NPT, NPT)])
    pltpu.sync_copy(den_s.at[pl.ds(tid * 640, 640)],
                    denp.at[pl.ds(c * NP + tid * 640, 640)])


@functools.lru_cache(maxsize=None)
def _score_kernel():
    mesh = plsc.VectorSubcoreMesh(core_axis_name="c", subcore_axis_name="s")
    return functools.partial(
        pl.kernel,
        mesh=mesh,
        out_type=jax.ShapeDtypeStruct((2 * LP, 16), _F32),
        scratch_types=[
            pltpu.VMEM((LBT, 128), _I32),
            pltpu.VMEM((LBT, 128), _I32),
            pltpu.VMEM((128, 128), _F32),
            pltpu.VMEM((128, 128), _F32),
            pltpu.VMEM((128, 128), _F32),
            pltpu.VMEM((128, 128), _F32),
            pltpu.VMEM((128, 16), _F32),
            pltpu.SemaphoreType.DMA,
            pltpu.SemaphoreType.DMA,
        ],
    )(_score_body)


def _score_body(u2w, item2, i2w, root, idx_a, idx_b, out,
                ia_v, ib_v, ra0, ra1, rb0, rb1, res_v, sem0, sem1):
    c = lax.axis_index("c")
    s = lax.axis_index("s")
    wid = s * 2 + c

    ra = (ra0, ra1)
    rb = (rb0, rb1)
    sem = (sem0, sem1)

    for r in range(2):
        tab_a = u2w if r == 0 else i2w
        tab_b = item2 if r == 0 else root
        pltpu.sync_copy(idx_a.at[r, wid], ia_v)
        pltpu.sync_copy(idx_b.at[r, wid], ib_v)

        def _gathers(bb, p, tab_a=tab_a, tab_b=tab_b):
            return ((tab_a.at[ia_v.at[bb]], ra[p], sem[p]),
                    (tab_b.at[ib_v.at[bb]], rb[p], sem[p]))

        def issue(bb, p, _g=_gathers):
            for a in _g(bb, p):
                pltpu.async_copy(*a)

        def drain(bb, p, _g=_gathers):
            for a in _g(bb, p):
                pltpu.make_async_copy(*a).wait()

        def process(bb, p, r=r):
            def e_body(e, carry, p=p):
                acc = ra[p][e, pl.ds(0, 16)] * rb[p][e, pl.ds(0, 16)]
                for k in range(1, 8):
                    acc = acc + (ra[p][e, pl.ds(k * 16, 16)] *
                                 rb[p][e, pl.ds(k * 16, 16)])
                res_v[e, :] = acc
                return carry

            lax.fori_loop(0, 128, e_body, 0)
            pltpu.sync_copy(
                res_v, out.at[pl.ds(r * LP + wid * LPT + bb * 128, 128)])

        issue(0, 0)

        def dbl_body(i, carry, issue=issue, drain=drain, process=process):
            b0 = i * 2
            drain(b0, 0)
            issue(b0 + 1, 1)
            process(b0, 0)
            drain(b0 + 1, 1)

            @pl.when(i < (LBT - 1) // 2)
            def _():
                issue(b0 + 2, 0)

            process(b0 + 1, 1)
            return carry

        lax.fori_loop(0, LBT // 2, dbl_body, 0)
        drain(LBT - 1, 0)
        process(LBT - 1, 0)


def _rowsum_body(x_ref, o_ref):
    s = jnp.sum(x_ref[...], axis=1, keepdims=True)
    o_ref[...] = jnp.broadcast_to(s, (s.shape[0], 8))


def _rowsum(x):
    rows, minor = x.shape
    rb = 512
    return pl.pallas_call(
        _rowsum_body,
        grid=(rows // rb,),
        in_specs=[pl.BlockSpec((rb, minor), lambda i: (i, 0))],
        out_specs=pl.BlockSpec((rb, 8), lambda i: (i, 0)),
        out_shape=jax.ShapeDtypeStruct((rows, 8), _F32),
    )(x)


def kernel(x_user, x_item, root_user,
           Wsrc1ui, Wdst1ui, asrc1ui, adst1ui, b1ui,
           Wsrc1iu, Wdst1iu, asrc1iu, adst1iu, b1iu,
           Wsrc2ui, Wdst2ui, asrc2ui, adst2ui, b2ui,
           Wsrc2iu, Wdst2iu, asrc2iu, adst2iu, b2iu,
           rel_weight,
           edge_index_ui, edge_index_iu,
           edge_label_index_ui, edge_label_index_iu):
    src = edge_index_ui[0].astype(_I32)
    dst = edge_index_ui[1].astype(_I32)
    src_f = jnp.concatenate([src, jnp.zeros((EP - E,), _I32)])
    dst_f = jnp.concatenate([dst, jnp.full((EP - E,), N, _I32)])
    dst_m = dst_f.reshape(-1, 128)

    hs1, ss1, sd1 = _prep1(x_user, x_item, Wsrc1ui, asrc1ui, Wdst1ui, adst1ui)
    zpad = jnp.zeros((NP - N,), _F32)
    nump1, denp1 = _edge_pass_kernel()(
        hs1, ss1[:, 0], jnp.concatenate([sd1[:, 0], zpad]),
        src_f, dst_m, dst_f)
    den1 = denp1.reshape(2, NP)[:, :N]
    nump1 = nump1[:, :N]

    hs2, ss2, sd2 = _comb_prep2(
        nump1[0], nump1[1], den1[0].reshape(N, 1), den1[1].reshape(N, 1),
        b1ui, root_user, Wsrc2ui, asrc2ui, Wdst2ui, adst2ui)
    nump2, denp2 = _edge_pass_kernel()(
        hs2, ss2[:, 0], jnp.concatenate([sd2[:, 0], zpad]),
        src_f, dst_m, dst_f)
    den2 = denp2.reshape(2, NP)[:, :N]
    nump2 = nump2[:, :N]

    item2, u2w, i2w = _final(
        nump2[0], nump2[1], den2[0].reshape(N, 1), den2[1].reshape(N, 1),
        b2ui, root_user, rel_weight[0].reshape(1, D), rel_weight[1].reshape(1, D))

    def _pad(a):
        return jnp.concatenate([a.astype(_I32), jnp.zeros((LP - L,), _I32)])

    idx_a = jnp.stack([_pad(edge_label_index_ui[0]),
                       _pad(edge_label_index_iu[0])]).reshape(2, NW, LBT, 128)
    idx_b = jnp.stack([_pad(edge_label_index_ui[1]),
                       _pad(edge_label_index_iu[1])]).reshape(2, NW, LBT, 128)

    prod = _score_kernel()(u2w, item2, i2w, root_user, idx_a, idx_b)
    preds = _rowsum(prod)[:, 0]
    return preds.reshape(2, LP)[:, :L]


# speedup vs baseline: 8.3509x; 1.0031x over previous
import functools

import jax
import jax.numpy as jnp
from jax import lax
from jax.experimental import pallas as pl
from jax.experimental.pallas import tpu as pltpu
from jax.experimental.pallas import tpu_sc as plsc

N = 10000
NP = 10240
NPS = 10112
NPT = NPS // 16
D = 128
E = 160000
NW = 32
EPT = 5120
EP = NW * EPT
NB = EPT // 128
L = 50000
LBT = 13
LPT = LBT * 128
LP = NW * LPT

_F32 = jnp.float32
_I32 = jnp.int32


_RB = 400
_GRID = N // _RB


def _dotT(a1, m):
    return lax.dot_general(a1, m, (((1,), (1,)), ((), ())),
                           preferred_element_type=_F32)


def _col_dot(x, a1):
    col = lax.dot_general(x, a1, (((1,), (1,)), ((), ())),
                          preferred_element_type=_F32)
    return jnp.broadcast_to(col, (col.shape[0], 8))


def _prep1_body(xs_ref, xd_ref, ws_ref, as_ref, wd_ref, ad_ref,
                hs_ref, ss_ref, sd_ref):
    hs = jnp.dot(xs_ref[...], ws_ref[...], preferred_element_type=_F32)
    hs_ref[...] = hs
    ss_ref[...] = _col_dot(hs, as_ref[...])
    wda = _dotT(ad_ref[...], wd_ref[...])
    sd_ref[...] = _col_dot(xd_ref[...], wda)


def _prep1(x_src, x_dst, ws, a_s, wd, a_d):
    full = pl.BlockSpec((128, 128), lambda i: (0, 0))
    vec = pl.BlockSpec((1, 128), lambda i: (0, 0))
    blk = pl.BlockSpec((_RB, 128), lambda i: (i, 0))
    row = pl.BlockSpec((_RB, 8), lambda i: (i, 0))
    return pl.pallas_call(
        _prep1_body,
        grid=(_GRID,),
        in_specs=[blk, blk, full, vec, full, vec],
        out_specs=[blk, row, row],
        out_shape=[jax.ShapeDtypeStruct((N, D), _F32),
                   jax.ShapeDtypeStruct((N, 8), _F32),
                   jax.ShapeDtypeStruct((N, 8), _F32)],
    )(x_src, x_dst, ws, a_s.reshape(1, D), wd, a_d.reshape(1, D))


def _comb_body(n0_ref, n1_ref, d0_ref, d1_ref, b_ref, root_ref,
               ws_ref, as_ref, wd_ref, ad_ref, hs_ref, ss_ref, sd_ref):
    den = d0_ref[...] + d1_ref[...]
    item1 = (n0_ref[...] + n1_ref[...]) / (den + 1e-16) + b_ref[...]
    item1 = jnp.maximum(item1, 0.0)
    u1 = jnp.maximum(root_ref[...], 0.0)
    hs = jnp.dot(u1, ws_ref[...], preferred_element_type=_F32)
    hs_ref[...] = hs
    ss_ref[...] = _col_dot(hs, as_ref[...])
    wda = _dotT(ad_ref[...], wd_ref[...])
    sd_ref[...] = _col_dot(item1, wda)


def _comb_prep2(n0, n1, d0, d1, b, root, ws, a_s, wd, a_d):
    full = pl.BlockSpec((128, 128), lambda i: (0, 0))
    vec = pl.BlockSpec((1, 128), lambda i: (0, 0))
    blk = pl.BlockSpec((_RB, 128), lambda i: (i, 0))
    col = pl.BlockSpec((_RB, 1), lambda i: (i, 0))
    row = pl.BlockSpec((_RB, 8), lambda i: (i, 0))
    return pl.pallas_call(
        _comb_body,
        grid=(_GRID,),
        in_specs=[blk, blk, col, col, vec, blk, full, vec, full, vec],
        out_specs=[blk, row, row],
        out_shape=[jax.ShapeDtypeStruct((N, D), _F32),
                   jax.ShapeDtypeStruct((N, 8), _F32),
                   jax.ShapeDtypeStruct((N, 8), _F32)],
    )(n0, n1, d0, d1, b.reshape(1, D), root, ws, a_s.reshape(1, D),
      wd, a_d.reshape(1, D))


def _final_body(n0_ref, n1_ref, d0_ref, d1_ref, b_ref, root_ref,
                rw0_ref, rw1_ref, item2_ref, u2w_ref, i2w_ref):
    den = d0_ref[...] + d1_ref[...]
    item2 = (n0_ref[...] + n1_ref[...]) / (den + 1e-16) + b_ref[...]
    item2_ref[...] = item2
    u2w_ref[...] = root_ref[...] * rw0_ref[...]
    i2w_ref[...] = item2 * rw1_ref[...]


def _final(n0, n1, d0, d1, b, root, rw0, rw1):
    vec = pl.BlockSpec((1, 128), lambda i: (0, 0))
    blk = pl.BlockSpec((_RB, 128), lambda i: (i, 0))
    col = pl.BlockSpec((_RB, 1), lambda i: (i, 0))
    return pl.pallas_call(
        _final_body,
        grid=(_GRID,),
        in_specs=[blk, blk, col, col, vec, blk, vec, vec],
        out_specs=[blk, blk, blk],
        out_shape=[jax.ShapeDtypeStruct((N, D), _F32),
                   jax.ShapeDtypeStruct((N, D), _F32),
                   jax.ShapeDtypeStruct((N, D), _F32)],
    )(n0, n1, d0, d1, b.reshape(1, D), root, rw0, rw1)


@functools.lru_cache(maxsize=None)
def _edge_pass_kernel():
    mesh = plsc.VectorSubcoreMesh(core_axis_name="c", subcore_axis_name="s")
    return functools.partial(
        pl.kernel,
        mesh=mesh,
        out_type=[jax.ShapeDtypeStruct((2, NPS, D), _F32),
                  jax.ShapeDtypeStruct((2 * NP,), _F32)],
        scratch_types=[
            pltpu.VMEM((128,), _F32),
            pltpu.VMEM((128,), _F32),
            pltpu.VMEM((128,), _F32),
            pltpu.VMEM((128,), _F32),
            pltpu.VMEM((EPT,), _I32),
            pltpu.VMEM((EPT,), _I32),
            pltpu.VMEM((NB, 128), _I32),
            pltpu.VMEM((128,), _F32),
            pltpu.VMEM((128,), _F32),
            pltpu.VMEM((128, 128), _F32),
            pltpu.VMEM((128, 128), _F32),
            pltpu.VMEM((640,), _F32),
            pltpu.VMEM_SHARED((NPS, D), _F32),
            pltpu.VMEM_SHARED((NP,), _F32),
            pltpu.SemaphoreType.DMA,
            pltpu.SemaphoreType.DMA,
            pltpu.SemaphoreType.DMA,
            pltpu.SemaphoreType.DMA,
        ],
    )(_edge_pass_body)


def _edge_pass_body(hs_hbm, ss_hbm, sd_hbm, srcf_hbm, dst_hbm, dstf_hbm,
                    nump, denp,
                    ssg0, ssg1, sdg0, sdg1, src_f, dst_f, dst_v, w0, w1,
                    rows0, rows1, zden_v, num_s, den_s,
                    sem0, sem1, ssem0, ssem1):
    c = lax.axis_index("c")
    s = lax.axis_index("s")
    wid = s * 2 + c
    tid = s

    ssg = (ssg0, ssg1)
    sdg = (sdg0, sdg1)
    rows = (rows0, rows1)
    w_v = (w0, w1)
    sem = (sem0, sem1)
    ssem = (ssem0, ssem1)

    pltpu.sync_copy(srcf_hbm.at[pl.ds(wid * EPT, EPT)], src_f)
    pltpu.sync_copy(dstf_hbm.at[pl.ds(wid * EPT, EPT)], dst_f)
    pltpu.sync_copy(dst_hbm.at[pl.ds(wid * NB, NB)], dst_v)

    z16 = jnp.zeros((16,), _F32)

    def zrow_body(r, carry):
        for k in range(8):
            rows0[r, pl.ds(k * 16, 16)] = z16
        return carry

    lax.fori_loop(0, 128, zrow_body, 0)
    for i in range(40):
        zden_v[pl.ds(i * 16, 16)] = z16

    for j in range(4):
        pltpu.sync_copy(rows0, num_s.at[pl.ds(tid * NPT + j * 128, 128)])
    pltpu.sync_copy(rows0.at[pl.ds(0, NPT - 512)],
                    num_s.at[pl.ds(tid * NPT + 512, NPT - 512)])
    pltpu.sync_copy(zden_v, den_s.at[pl.ds(tid * 640, 640)])
    plsc.subcore_barrier()

    def _gathers(b, p):
        bidx = src_f.at[pl.ds(b * 128, 128)]
        didx = dst_f.at[pl.ds(b * 128, 128)]
        return ((hs_hbm.at[bidx], rows[p], sem[p]),
                (ss_hbm.at[bidx], ssg[p], sem[p]),
                (sd_hbm.at[didx], sdg[p], sem[p]))

    def issue(b, p):
        for a in _gathers(b, p):
            pltpu.async_copy(*a)

    def drain(b, p):
        for a in _gathers(b, p):
            pltpu.make_async_copy(*a).wait()

    def _scatters(b, p):
        drow = dst_v.at[b]
        return ((w_v[p], den_s.at[drow]),
                (rows[p], num_s.at[drow]))

    def compute(b, p):
        def g_body(g, carry, p=p):
            x = ssg[p][pl.ds(g * 16, 16)] + sdg[p][pl.ds(g * 16, 16)]
            w16 = jnp.exp(jnp.maximum(x, x * 0.2))
            w_v[p][pl.ds(g * 16, 16)] = w16
            for el in range(16):
                e = g * 16 + el
                wv = jnp.full((16,), w16[el], _F32)
                for k in range(8):
                    rows[p][e, pl.ds(k * 16, 16)] = (
                        rows[p][e, pl.ds(k * 16, 16)] * wv)
            return carry

        lax.fori_loop(0, 8, g_body, 0)

    def scatter(b, p):
        for a in _scatters(b, p):
            pltpu.sync_copy(a[0], a[1], add=True)

    issue(0, 0)

    def dbl_body(i, carry):
        b0 = i * 2
        drain(b0, 0)
        issue(b0 + 1, 1)
        compute(b0, 0)
        scatter(b0, 0)
        drain(b0 + 1, 1)

        @pl.when(i < NB // 2 - 1)
        def _():
            issue(b0 + 2, 0)

        compute(b0 + 1, 1)
        scatter(b0 + 1, 1)
        return carry

    lax.fori_loop(0, NB // 2, dbl_body, 0)
    plsc.subcore_barrier()

    pltpu.sync_copy(num_s.at[pl.ds(tid * NPT, NPT)],
                    nump.at[c, pl.ds(tid * NPT, NPT)])
    pltpu.sync_copy(den_s.at[pl.ds(tid * 640, 640)],
                    denp.at[pl.ds(c * NP + tid * 640, 640)])


@functools.lru_cache(maxsize=None)
def _score_kernel():
    mesh = plsc.VectorSubcoreMesh(core_axis_name="c", subcore_axis_name="s")
    return functools.partial(
        pl.kernel,
        mesh=mesh,
        out_type=jax.ShapeDtypeStruct((2 * LP, 16), _F32),
        scratch_types=[
            pltpu.VMEM((LBT, 128), _I32),
            pltpu.VMEM((LBT, 128), _I32),
            pltpu.VMEM((128, 128), _F32),
            pltpu.VMEM((128, 128), _F32),
            pltpu.VMEM((128, 128), _F32),
            pltpu.VMEM((128, 128), _F32),
            pltpu.VMEM((128, 128), _F32),
            pltpu.VMEM((128, 128), _F32),
            pltpu.VMEM((128, 16), _F32),
            pltpu.SemaphoreType.DMA,
            pltpu.SemaphoreType.DMA,
            pltpu.SemaphoreType.DMA,
        ],
    )(_score_body)


def _score_body(u2w, item2, i2w, root, idx_a, idx_b, out,
                ia_v, ib_v, ra0, ra1, ra2, rb0, rb1, rb2, res_v,
                sem0, sem1, sem2):
    c = lax.axis_index("c")
    s = lax.axis_index("s")
    wid = s * 2 + c

    ra = (ra0, ra1, ra2)
    rb = (rb0, rb1, rb2)
    sem = (sem0, sem1, sem2)

    for r in range(2):
        tab_a = u2w if r == 0 else i2w
        tab_b = item2 if r == 0 else root
        pltpu.sync_copy(idx_a.at[r, wid], ia_v)
        pltpu.sync_copy(idx_b.at[r, wid], ib_v)

        def _gathers(bb, p, tab_a=tab_a, tab_b=tab_b):
            return ((tab_a.at[ia_v.at[bb]], ra[p], sem[p]),
                    (tab_b.at[ib_v.at[bb]], rb[p], sem[p]))

        def issue(bb, p, _g=_gathers):
            for a in _g(bb, p):
                pltpu.async_copy(*a)

        def drain(bb, p, _g=_gathers):
            for a in _g(bb, p):
                pltpu.make_async_copy(*a).wait()

        def process(bb, p, r=r):
            def e_body(e, carry, p=p):
                acc = ra[p][e, pl.ds(0, 16)] * rb[p][e, pl.ds(0, 16)]
                for k in range(1, 8):
                    acc = acc + (ra[p][e, pl.ds(k * 16, 16)] *
                                 rb[p][e, pl.ds(k * 16, 16)])
                res_v[e, :] = acc
                return carry

            lax.fori_loop(0, 128, e_body, 0)
            pltpu.sync_copy(
                res_v, out.at[pl.ds(r * LP + wid * LPT + bb * 128, 128)])

        issue(0, 0)
        issue(1, 1)

        def tri_body(i3, carry, issue=issue, drain=drain, process=process):
            for j in range(3):
                b = i3 * 3 + j
                drain(b, j)

                @pl.when(b + 2 < LBT)
                def _(b=b, j=j, issue=issue):
                    issue(b + 2, (j + 2) % 3)

                process(b, j)
            return carry

        lax.fori_loop(0, (LBT - 1) // 3, tri_body, 0)
        drain(LBT - 1, 0)
        process(LBT - 1, 0)


def _rowsum_body(x_ref, o_ref):
    s = jnp.sum(x_ref[...], axis=1, keepdims=True)
    o_ref[...] = jnp.broadcast_to(s, (s.shape[0], 8))


def _rowsum(x):
    rows, minor = x.shape
    rb = 512
    return pl.pallas_call(
        _rowsum_body,
        grid=(rows // rb,),
        in_specs=[pl.BlockSpec((rb, minor), lambda i: (i, 0))],
        out_specs=pl.BlockSpec((rb, 8), lambda i: (i, 0)),
        out_shape=jax.ShapeDtypeStruct((rows, 8), _F32),
    )(x)


def kernel(x_user, x_item, root_user,
           Wsrc1ui, Wdst1ui, asrc1ui, adst1ui, b1ui,
           Wsrc1iu, Wdst1iu, asrc1iu, adst1iu, b1iu,
           Wsrc2ui, Wdst2ui, asrc2ui, adst2ui, b2ui,
           Wsrc2iu, Wdst2iu, asrc2iu, adst2iu, b2iu,
           rel_weight,
           edge_index_ui, edge_index_iu,
           edge_label_index_ui, edge_label_index_iu):
    src = edge_index_ui[0].astype(_I32)
    dst = edge_index_ui[1].astype(_I32)
    src_f = jnp.concatenate([src, jnp.zeros((EP - E,), _I32)])
    dst_f = jnp.concatenate([dst, jnp.full((EP - E,), N, _I32)])
    dst_m = dst_f.reshape(-1, 128)

    hs1, ss1, sd1 = _prep1(x_user, x_item, Wsrc1ui, asrc1ui, Wdst1ui, adst1ui)
    zpad = jnp.zeros((NP - N,), _F32)
    nump1, denp1 = _edge_pass_kernel()(
        hs1, ss1[:, 0], jnp.concatenate([sd1[:, 0], zpad]),
        src_f, dst_m, dst_f)
    den1 = denp1.reshape(2, NP)[:, :N]
    nump1 = nump1[:, :N]

    hs2, ss2, sd2 = _comb_prep2(
        nump1[0], nump1[1], den1[0].reshape(N, 1), den1[1].reshape(N, 1),
        b1ui, root_user, Wsrc2ui, asrc2ui, Wdst2ui, adst2ui)
    nump2, denp2 = _edge_pass_kernel()(
        hs2, ss2[:, 0], jnp.concatenate([sd2[:, 0], zpad]),
        src_f, dst_m, dst_f)
    den2 = denp2.reshape(2, NP)[:, :N]
    nump2 = nump2[:, :N]

    item2, u2w, i2w = _final(
        nump2[0], nump2[1], den2[0].reshape(N, 1), den2[1].reshape(N, 1),
        b2ui, root_user, rel_weight[0].reshape(1, D), rel_weight[1].reshape(1, D))

    def _pad(a):
        return jnp.concatenate([a.astype(_I32), jnp.zeros((LP - L,), _I32)])

    idx_a = jnp.stack([_pad(edge_label_index_ui[0]),
                       _pad(edge_label_index_iu[0])]).reshape(2, NW, LBT, 128)
    idx_b = jnp.stack([_pad(edge_label_index_ui[1]),
                       _pad(edge_label_index_iu[1])]).reshape(2, NW, LBT, 128)

    prod = _score_kernel()(u2w, item2, i2w, root_user, idx_a, idx_b)
    preds = _rowsum(prod)[:, 0]
    return preds.reshape(2, LP)[:, :L]
